# baseline (speedup 1.0000x reference)
import dataclasses
import functools

import jax
import jax.numpy as jnp
from jax import lax
from jax.experimental import pallas as pl
from jax.experimental.pallas import tpu as pltpu
from jax.experimental.pallas import tpu_sc as plsc

NC = 2
NS = 16
L = 16
NW = NC * NS


@functools.lru_cache(maxsize=None)
def _bag_call(N, T, V, D):
    rows_w = N // NW
    chars_w = rows_w * T
    rpc = 4
    cpc = rpc * T
    n_chunks = rows_w // rpc
    vecs_per_chunk = cpc // L
    n_maskvec = chars_w // L

    mesh = plsc.VectorSubcoreMesh(core_axis_name="c", subcore_axis_name="s")
    cp = pltpu.CompilerParams()
    if "needs_layout_passes" in pltpu.CompilerParams.__dataclass_fields__:
        cp = dataclasses.replace(cp, needs_layout_passes=False)
    if "use_tc_tiling_on_sc" in pltpu.CompilerParams.__dataclass_fields__:
        cp = dataclasses.replace(cp, use_tc_tiling_on_sc=False)

    @functools.partial(
        pl.kernel,
        out_type=jax.ShapeDtypeStruct((N, D), jnp.float32),
        mesh=mesh,
        compiler_params=cp,
        scratch_types=[
            pltpu.VMEM((chars_w,), jnp.int32),
            pltpu.VMEM((rows_w,), jnp.int32),
            pltpu.VMEM((n_chunks, cpc), jnp.int32),
            pltpu.VMEM((cpc, D), jnp.float32),
            pltpu.VMEM((cpc, D), jnp.float32),
            pltpu.VMEM((rows_w, D), jnp.float32),
            pltpu.SemaphoreType.DMA,
            pltpu.SemaphoreType.DMA,
        ],
    )
    def bag(chars_hbm, len_hbm, table_hbm, out_hbm,
            chars_v, len_v, idx_v, g0, g1, outb, s0, s1):
        wid = lax.axis_index("s") * NC + lax.axis_index("c")
        cbase = wid * chars_w
        rbase = wid * rows_w
        pltpu.sync_copy(chars_hbm.at[pl.ds(cbase, chars_w)], chars_v)
        pltpu.sync_copy(len_hbm.at[pl.ds(rbase, rows_w)], len_v)

        iota = lax.iota(jnp.int32, L)

        @pl.loop(0, n_maskvec)
        def _(jm):
            pos = jm * L + iota
            row = pos // T
            t = pos - row * T
            lenv = plsc.load_gather(len_v, [row])
            c = chars_v[pl.ds(jm * L, L)]
            cm = jnp.where(t < lenv, c, jnp.zeros_like(c))
            idx_v[jm // vecs_per_chunk,
                  pl.ds((jm % vecs_per_chunk) * L, L)] = cm

        def gcopy(j, buf, sem):
            return pltpu.make_async_copy(table_hbm.at[idx_v.at[j]], buf, sem)

        def acc_chunk(j, buf):
            for r in range(rpc):
                row = j * rpc + r
                for v in range(D // L):
                    sl = pl.ds(v * L, L)
                    acc = buf[r * T, sl]
                    for t in range(1, T):
                        acc = acc + buf[r * T + t, sl]
                    outb[row, sl] = acc

        gcopy(0, g0, s0).start()
        gcopy(1, g1, s1).start()

        @pl.loop(0, n_chunks, step=2)
        def _(j):
            gcopy(j, g0, s0).wait()
            acc_chunk(j, g0)

            @pl.when(j + 2 < n_chunks)
            def _():
                gcopy(j + 2, g0, s0).start()

            gcopy(j + 1, g1, s1).wait()
            acc_chunk(j + 1, g1)

            @pl.when(j + 3 < n_chunks)
            def _():
                gcopy(j + 3, g1, s1).start()

        pltpu.sync_copy(outb, out_hbm.at[pl.ds(rbase, rows_w)])

    return bag


@functools.lru_cache(maxsize=None)
def _proj_call(N, D, H, blk):
    def body(x_ref, len_ref, w_ref, o_ref):
        denom = jnp.maximum(len_ref[...].astype(jnp.float32), 1.0)
        o_ref[...] = jnp.dot(x_ref[...] / denom, w_ref[...],
                             preferred_element_type=jnp.float32)

    return pl.pallas_call(
        body,
        grid=(N // blk,),
        in_specs=[
            pl.BlockSpec((blk, D), lambda i: (i, 0)),
            pl.BlockSpec((blk, 1), lambda i: (i, 0)),
            pl.BlockSpec((D, H), lambda i: (0, 0)),
        ],
        out_specs=pl.BlockSpec((blk, H), lambda i: (i, 0)),
        out_shape=jax.ShapeDtypeStruct((N, H), jnp.float32),
    )


def kernel(char_input, lengths, emb, W):
    B, S, T = char_input.shape
    N = B * S
    V, D = emb.shape
    H = W.shape[1]

    chars = char_input.reshape(N * T).astype(jnp.int32)
    flat_len = lengths.reshape(N)
    len_i32 = flat_len.astype(jnp.int32)

    bag = _bag_call(N, T, V, D)
    sums = bag(chars, len_i32, emb)

    out = _proj_call(N, D, H, 1024)(sums, len_i32.reshape(N, 1), W)
    return out.reshape(B, S, H), flat_len


# speedup vs baseline: 7.1544x; 7.1544x over previous
import dataclasses
import functools

import jax
import jax.numpy as jnp
from jax import lax
from jax.experimental import pallas as pl
from jax.experimental.pallas import tpu as pltpu
from jax.experimental.pallas import tpu_sc as plsc

NC = 2
NS = 16
L = 16
NW = NC * NS


@functools.lru_cache(maxsize=None)
def _bag_call(N, T, V, D):
    rows_w = N // NW
    n_groups = rows_w // L

    mesh = plsc.VectorSubcoreMesh(core_axis_name="c", subcore_axis_name="s")
    cp = pltpu.CompilerParams()
    if "needs_layout_passes" in pltpu.CompilerParams.__dataclass_fields__:
        cp = dataclasses.replace(cp, needs_layout_passes=False)
    if "use_tc_tiling_on_sc" in pltpu.CompilerParams.__dataclass_fields__:
        cp = dataclasses.replace(cp, use_tc_tiling_on_sc=False)

    @functools.partial(
        pl.kernel,
        out_type=jax.ShapeDtypeStruct((NW, D, rows_w), jnp.float32),
        mesh=mesh,
        compiler_params=cp,
        scratch_types=[
            pltpu.VMEM((T, rows_w), jnp.int32),
            pltpu.VMEM((rows_w,), jnp.int32),
            pltpu.VMEM((V * D,), jnp.float32),
            pltpu.VMEM((D, rows_w), jnp.float32),
        ],
    )
    def bag(charsT_hbm, len_hbm, table_hbm, out_hbm,
            charsT_v, len_v, table_v, outT_v):
        wid = lax.axis_index("s") * NC + lax.axis_index("c")
        rbase = wid * rows_w
        pltpu.sync_copy(charsT_hbm.at[wid], charsT_v)
        pltpu.sync_copy(len_hbm.at[pl.ds(rbase, rows_w)], len_v)
        pltpu.sync_copy(table_hbm, table_v)

        @pl.loop(0, n_groups)
        def _(g):
            sl = pl.ds(g * L, L)
            lenv = len_v[sl]
            invl = 1.0 / jnp.maximum(lenv.astype(jnp.float32), 1.0)
            cms = []
            for t in range(T):
                c = charsT_v[t, sl]
                cms.append(jnp.where(t < lenv, c, jnp.zeros_like(c)) * D)
            for d in range(D):
                acc = plsc.load_gather(table_v, [cms[0] + d])
                for t in range(1, T):
                    acc = acc + plsc.load_gather(table_v, [cms[t] + d])
                outT_v[d, sl] = acc * invl

        pltpu.sync_copy(outT_v, out_hbm.at[wid])

    return bag


@functools.lru_cache(maxsize=None)
def _proj_call(NB, D, R, H):
    def body(xt_ref, w_ref, o_ref):
        o_ref[...] = lax.dot_general(
            xt_ref[0], w_ref[...],
            dimension_numbers=(((0,), (0,)), ((), ())),
            preferred_element_type=jnp.float32,
        )

    return pl.pallas_call(
        body,
        grid=(NB,),
        in_specs=[
            pl.BlockSpec((1, D, R), lambda i: (i, 0, 0)),
            pl.BlockSpec((D, H), lambda i: (0, 0)),
        ],
        out_specs=pl.BlockSpec((R, H), lambda i: (i, 0)),
        out_shape=jax.ShapeDtypeStruct((NB * R, H), jnp.float32),
    )


def kernel(char_input, lengths, emb, W):
    B, S, T = char_input.shape
    N = B * S
    V, D = emb.shape
    H = W.shape[1]
    rows_w = N // NW

    charsT = (char_input.reshape(NW, rows_w, T)
              .transpose(0, 2, 1)
              .astype(jnp.int32))
    flat_len = lengths.reshape(N)
    len_i32 = flat_len.astype(jnp.int32)
    table_flat = emb.reshape(V * D)

    pooledT = _bag_call(N, T, V, D)(charsT, len_i32, table_flat)
    out = _proj_call(NW, D, rows_w, H)(pooledT, W)
    return out.reshape(B, S, H), flat_len


# speedup vs baseline: 24.6532x; 3.4459x over previous
import dataclasses
import functools

import jax
import jax.numpy as jnp
from jax import lax
from jax.experimental import pallas as pl
from jax.experimental.pallas import tpu as pltpu
from jax.experimental.pallas import tpu_sc as plsc

NC = 2
NS = 16
L = 16
NW = NC * NS
REP = 8


@functools.lru_cache(maxsize=None)
def _bag_call(N, T, V, D):
    rows_w = N // NW
    n_groups = rows_w // L
    pairs = D // 2
    vpad = -(-V // 8) * 8
    sub = vpad * REP

    mesh = plsc.VectorSubcoreMesh(core_axis_name="c", subcore_axis_name="s")
    cp = pltpu.CompilerParams()
    if "needs_layout_passes" in pltpu.CompilerParams.__dataclass_fields__:
        cp = dataclasses.replace(cp, needs_layout_passes=False)
    if "use_tc_tiling_on_sc" in pltpu.CompilerParams.__dataclass_fields__:
        cp = dataclasses.replace(cp, use_tc_tiling_on_sc=False)

    @functools.partial(
        pl.kernel,
        out_type=jax.ShapeDtypeStruct((NW, D, rows_w), jnp.float32),
        mesh=mesh,
        compiler_params=cp,
        scratch_types=[
            pltpu.VMEM((T, rows_w), jnp.int32),
            pltpu.VMEM((rows_w,), jnp.int32),
            pltpu.VMEM((pairs * sub,), jnp.int32),
            pltpu.VMEM((D, rows_w), jnp.float32),
        ],
    )
    def bag(charsT_hbm, len_hbm, table_hbm, out_hbm,
            charsT_v, len_v, table_v, outT_v):
        wid = lax.axis_index("s") * NC + lax.axis_index("c")
        rbase = wid * rows_w
        pltpu.sync_copy(charsT_hbm.at[wid], charsT_v)
        pltpu.sync_copy(len_hbm.at[pl.ds(rbase, rows_w)], len_v)
        pltpu.sync_copy(table_hbm, table_v)

        iota = lax.iota(jnp.int32, L)
        lane_rep = jnp.bitwise_and(iota, REP - 1)
        himask = jnp.full((L,), -65536, jnp.int32)

        @pl.loop(0, n_groups)
        def _(g):
            sl = pl.ds(g * L, L)
            lenv = len_v[sl]
            invl = 1.0 / jnp.maximum(lenv.astype(jnp.float32), 1.0)
            cms = []
            for t in range(T):
                c = charsT_v[t, sl]
                cm = jnp.where(t < lenv, c, jnp.zeros_like(c))
                cms.append(cm * REP + lane_rep)
            for p in range(pairs):
                tv = table_v.at[pl.ds(p * sub, sub)]
                w0 = plsc.load_gather(tv, [cms[0]])
                acc_lo = plsc.bitcast(jnp.left_shift(w0, 16), jnp.float32)
                acc_hi = plsc.bitcast(jnp.bitwise_and(w0, himask), jnp.float32)
                for t in range(1, T):
                    w = plsc.load_gather(tv, [cms[t]])
                    acc_lo = acc_lo + plsc.bitcast(
                        jnp.left_shift(w, 16), jnp.float32)
                    acc_hi = acc_hi + plsc.bitcast(
                        jnp.bitwise_and(w, himask), jnp.float32)
                outT_v[2 * p, sl] = acc_lo * invl
                outT_v[2 * p + 1, sl] = acc_hi * invl

        pltpu.sync_copy(outT_v, out_hbm.at[wid])

    return bag


@functools.lru_cache(maxsize=None)
def _proj_call(NB, D, R, H):
    def body(xt_ref, w_ref, o_ref):
        o_ref[...] = lax.dot_general(
            xt_ref[...], w_ref[...],
            dimension_numbers=(((1,), (0,)), ((), ())),
            preferred_element_type=jnp.float32,
        )

    return pl.pallas_call(
        body,
        in_specs=[
            pl.BlockSpec((NB, D, R), lambda: (0, 0, 0)),
            pl.BlockSpec((D, H), lambda: (0, 0)),
        ],
        out_specs=pl.BlockSpec((NB, R, H), lambda: (0, 0, 0)),
        out_shape=jax.ShapeDtypeStruct((NB, R, H), jnp.float32),
    )


def _pack_table(emb):
    V, D = emb.shape
    vpad = -(-V // 8) * 8
    pairs_bf = emb.astype(jnp.bfloat16).reshape(V, D // 2, 2)
    packed = lax.bitcast_convert_type(pairs_bf, jnp.uint32)
    packed = packed.T
    packed = jnp.pad(packed, ((0, 0), (0, vpad - V)))
    packed = jnp.repeat(packed[:, :, None], REP, axis=2)
    return lax.bitcast_convert_type(packed, jnp.int32).reshape(-1)


def kernel(char_input, lengths, emb, W):
    B, S, T = char_input.shape
    N = B * S
    V, D = emb.shape
    H = W.shape[1]
    rows_w = N // NW

    charsT = (char_input.reshape(NW, rows_w, T)
              .transpose(0, 2, 1)
              .astype(jnp.int32))
    flat_len = lengths.reshape(N)
    len_i32 = flat_len.astype(jnp.int32)

    pooledT = _bag_call(N, T, V, D)(charsT, len_i32, _pack_table(emb))
    out = _proj_call(NW, D, rows_w, H)(pooledT, W)
    return out.reshape(B, S, H), flat_len


# speedup vs baseline: 27.5327x; 1.1168x over previous
import dataclasses
import functools

import jax
import jax.numpy as jnp
from jax import lax
from jax.experimental import pallas as pl
from jax.experimental.pallas import tpu as pltpu
from jax.experimental.pallas import tpu_sc as plsc

NC = 2
NS = 16
L = 16
NW = NC * NS
REP = 8


@functools.lru_cache(maxsize=None)
def _bag_call(N, T, V, D):
    rows_w = N // NW
    n_groups = rows_w // L
    pairs = D // 2
    vpad = -(-V // 8) * 8
    sub = vpad * REP

    mesh = plsc.VectorSubcoreMesh(core_axis_name="c", subcore_axis_name="s")
    cp = pltpu.CompilerParams()
    if "needs_layout_passes" in pltpu.CompilerParams.__dataclass_fields__:
        cp = dataclasses.replace(cp, needs_layout_passes=False)
    if "use_tc_tiling_on_sc" in pltpu.CompilerParams.__dataclass_fields__:
        cp = dataclasses.replace(cp, use_tc_tiling_on_sc=False)

    @functools.partial(
        pl.kernel,
        out_type=jax.ShapeDtypeStruct((NW, D, rows_w), jnp.float32),
        mesh=mesh,
        compiler_params=cp,
        scratch_types=[
            pltpu.VMEM((T, rows_w), jnp.int32),
            pltpu.VMEM((rows_w,), jnp.int32),
            pltpu.VMEM((pairs * sub,), jnp.int32),
            pltpu.VMEM((D, rows_w), jnp.float32),
        ],
    )
    def bag(charsT_hbm, len_hbm, table_hbm, out_hbm,
            charsT_v, len_v, table_v, outT_v):
        wid = lax.axis_index("s") * NC + lax.axis_index("c")
        rbase = wid * rows_w
        pltpu.sync_copy(charsT_hbm.at[wid], charsT_v)
        pltpu.sync_copy(len_hbm.at[pl.ds(rbase, rows_w)], len_v)
        pltpu.sync_copy(table_hbm, table_v)

        iota = lax.iota(jnp.int32, L)
        lane_rep = jnp.bitwise_and(iota, REP - 1)
        himask = jnp.full((L,), -65536, jnp.int32)

        @pl.loop(0, n_groups)
        def _(g):
            sl = pl.ds(g * L, L)
            lenv = len_v[sl]
            invl = 1.0 / jnp.maximum(lenv.astype(jnp.float32), 1.0)
            cms = []
            for t in range(T):
                c = charsT_v[t, sl]
                cm = jnp.where(t < lenv, c, jnp.zeros_like(c))
                cms.append(cm * REP + lane_rep)
            for p in range(pairs):
                tv = table_v.at[pl.ds(p * sub, sub)]
                ws = [plsc.load_gather(tv, [cms[t]]) for t in range(T)]
                sums = [
                    plsc.bitcast(
                        plsc.bitcast(ws[k], jnp.bfloat16)
                        + plsc.bitcast(ws[k + 1], jnp.bfloat16),
                        jnp.int32)
                    for k in range(0, T - 1, 2)
                ]
                if T % 2:
                    sums.append(ws[-1])
                los = [plsc.bitcast(jnp.left_shift(s, 16), jnp.float32)
                       for s in sums]
                his = [plsc.bitcast(jnp.bitwise_and(s, himask), jnp.float32)
                       for s in sums]

                def _tree(vs):
                    while len(vs) > 1:
                        nxt = [vs[i] + vs[i + 1]
                               for i in range(0, len(vs) - 1, 2)]
                        if len(vs) % 2:
                            nxt.append(vs[-1])
                        vs = nxt
                    return vs[0]

                outT_v[2 * p, sl] = _tree(los) * invl
                outT_v[2 * p + 1, sl] = _tree(his) * invl

        pltpu.sync_copy(outT_v, out_hbm.at[wid])

    return bag


@functools.lru_cache(maxsize=None)
def _proj_call(NB, D, R, H):
    def body(xt_ref, w_ref, o_ref):
        o_ref[...] = lax.dot_general(
            xt_ref[...], w_ref[...],
            dimension_numbers=(((1,), (0,)), ((), ())),
            preferred_element_type=jnp.float32,
        )

    return pl.pallas_call(
        body,
        in_specs=[
            pl.BlockSpec((NB, D, R), lambda: (0, 0, 0)),
            pl.BlockSpec((D, H), lambda: (0, 0)),
        ],
        out_specs=pl.BlockSpec((NB, R, H), lambda: (0, 0, 0)),
        out_shape=jax.ShapeDtypeStruct((NB, R, H), jnp.float32),
    )


def _pack_table(emb):
    V, D = emb.shape
    vpad = -(-V // 8) * 8
    pairs_bf = emb.astype(jnp.bfloat16).reshape(V, D // 2, 2)
    packed = lax.bitcast_convert_type(pairs_bf, jnp.uint32)
    packed = packed.T
    packed = jnp.pad(packed, ((0, 0), (0, vpad - V)))
    packed = jnp.repeat(packed[:, :, None], REP, axis=2)
    return lax.bitcast_convert_type(packed, jnp.int32).reshape(-1)


def kernel(char_input, lengths, emb, W):
    B, S, T = char_input.shape
    N = B * S
    V, D = emb.shape
    H = W.shape[1]
    rows_w = N // NW

    charsT = (char_input.reshape(NW, rows_w, T)
              .transpose(0, 2, 1)
              .astype(jnp.int32))
    flat_len = lengths.reshape(N)
    len_i32 = flat_len.astype(jnp.int32)

    pooledT = _bag_call(N, T, V, D)(charsT, len_i32, _pack_table(emb))
    out = _proj_call(NW, D, rows_w, H)(pooledT, W)
    return out.reshape(B, S, H), flat_len
